# Initial kernel scaffold; baseline (speedup 1.0000x reference)
import jax
import jax.numpy as jnp
from jax.experimental import pallas as pl


def kernel(x, edge_index, edge_weight, W1, b1, W2, b2, Wl1, bl1, Wl2, bl2, Wl3, bl3):
    raise NotImplementedError("write your pallas kernel here")


# speedup vs baseline: 7.9762x; 7.9762x over previous
import functools

import jax
import jax.numpy as jnp
from jax import lax
from jax.experimental import pallas as pl
from jax.experimental.pallas import tpu as pltpu
from jax.experimental.pallas import tpu_sc as plsc

N = 10000
NP = 10240
E = 320000
NC, NS, L = 2, 16, 16
B = 128
D = 128
EP = 323584
CH_FEAT = EP // NS
CH_EDGE = EP // (NC * NS)
NB_FEAT = CH_FEAT // B
NB_EDGE = CH_EDGE // B
HALF = NP // NC
PTN = HALF // NS
RPT = NP // NS // B

_f32 = jnp.float32
_i32 = jnp.int32


def _mesh():
    return plsc.VectorSubcoreMesh(
        core_axis_name="c", subcore_axis_name="s", num_cores=NC, num_subcores=NS
    )


def _rsqrt16(v):
    i = lax.bitcast_convert_type(v, _i32)
    i = jnp.int32(0x5F3759DF) - lax.shift_right_arithmetic(i, 1)
    y = lax.bitcast_convert_type(i, _f32)
    for _ in range(4):
        y = y * (1.5 - 0.5 * v * y * y)
    return y


@functools.partial(
    pl.kernel,
    out_type=(
        jax.ShapeDtypeStruct((NP,), _f32),
        jax.ShapeDtypeStruct((NP, 16), _f32),
    ),
    mesh=_mesh(),
    compiler_params=pltpu.CompilerParams(needs_layout_passes=False),
    scratch_types=[
        pltpu.VMEM((B,), _i32),
        pltpu.VMEM((B,), _f32),
        pltpu.VMEM((NP,), _f32),
        pltpu.VMEM((PTN,), _f32),
        pltpu.VMEM((PTN,), _f32),
        pltpu.VMEM((PTN,), _f32),
        pltpu.VMEM((PTN, 16), _f32),
        pltpu.VMEM_SHARED((NS * NP,), _f32),
    ],
)
def _deg_dinv(dst_hbm, ew_hbm, dinv_hbm, d2_hbm,
              didx, ewv, dacc, tmpv, degv, dinvv, d2v, shp):
    c = lax.axis_index("c")
    s = lax.axis_index("s")

    def _z(j, _):
        dacc[pl.ds(j * L, L)] = jnp.zeros((L,), _f32)
        return 0

    lax.fori_loop(0, NP // L, _z, 0)

    def _body(b, _):
        base = s * CH_FEAT + b * B
        pltpu.sync_copy(dst_hbm.at[pl.ds(base, B)], didx)
        pltpu.sync_copy(ew_hbm.at[pl.ds(base, B)], ewv)

        def _sc(j, _):
            sl = pl.ds(j * L, L)
            plsc.addupdate_scatter(dacc, [didx[sl]], ewv[sl])
            return 0

        lax.fori_loop(0, B // L, _sc, 0)
        return 0

    lax.fori_loop(0, NB_FEAT, _body, 0)
    pltpu.sync_copy(dacc, shp.at[pl.ds(s * NP, NP)])
    plsc.subcore_barrier()

    gbase = c * HALF + s * PTN

    def _zz(j, _):
        degv[pl.ds(j * L, L)] = jnp.zeros((L,), _f32)
        return 0

    lax.fori_loop(0, PTN // L, _zz, 0)
    for r in range(NS):
        pltpu.sync_copy(shp.at[pl.ds(r * NP + gbase, PTN)], tmpv)

        def _acc(j, _):
            sl = pl.ds(j * L, L)
            degv[sl] = degv[sl] + tmpv[sl]
            return 0

        lax.fori_loop(0, PTN // L, _acc, 0)

    def _dinv(j, _):
        v = degv[pl.ds(j * L, L)] + 1.0
        dinvv[pl.ds(j * L, L)] = _rsqrt16(v)
        return 0

    lax.fori_loop(0, PTN // L, _dinv, 0)

    def _splat(i, _):
        y = plsc.load_gather(dinvv, [jnp.broadcast_to(i, (L,))])
        d2v[i] = y * y
        return 0

    lax.fori_loop(0, PTN, _splat, 0)
    pltpu.sync_copy(dinvv, dinv_hbm.at[pl.ds(gbase, PTN)])
    pltpu.sync_copy(d2v, d2_hbm.at[pl.ds(gbase, PTN)])


@functools.partial(
    pl.kernel,
    out_type=jax.ShapeDtypeStruct((EP,), _f32),
    mesh=_mesh(),
    compiler_params=pltpu.CompilerParams(needs_layout_passes=False),
    scratch_types=[
        pltpu.VMEM((NP,), _f32),
        pltpu.VMEM((B,), _i32),
        pltpu.VMEM((B,), _i32),
        pltpu.VMEM((B,), _f32),
        pltpu.VMEM((B,), _f32),
    ],
)
def _norm(src_hbm, dst_hbm, ew_hbm, dinv_hbm, norm_hbm, dv, sidx, didx, ewv, nmv):
    c = lax.axis_index("c")
    s = lax.axis_index("s")
    wid = s * NC + c
    pltpu.sync_copy(dinv_hbm, dv)
    base0 = wid * CH_EDGE

    def _body(b, _):
        base = base0 + b * B
        pltpu.sync_copy(src_hbm.at[pl.ds(base, B)], sidx)
        pltpu.sync_copy(dst_hbm.at[pl.ds(base, B)], didx)
        pltpu.sync_copy(ew_hbm.at[pl.ds(base, B)], ewv)

        def _g(j, _):
            sl = pl.ds(j * L, L)
            nm = plsc.load_gather(dv, [sidx[sl]]) * ewv[sl] * plsc.load_gather(dv, [didx[sl]])
            nmv[sl] = nm
            return 0

        lax.fori_loop(0, B // L, _g, 0)
        pltpu.sync_copy(nmv, norm_hbm.at[pl.ds(base, B)])
        return 0

    lax.fori_loop(0, NB_EDGE, _body, 0)


def _make_agg(feat_split):
    nvr = D // L
    nb = NB_FEAT if feat_split else NB_EDGE

    @functools.partial(
        pl.kernel,
        out_type=jax.ShapeDtypeStruct((NC, NP, D), _f32),
        mesh=_mesh(),
        compiler_params=pltpu.CompilerParams(needs_layout_passes=False),
        scratch_types=[
            pltpu.VMEM((B,), _i32),
            pltpu.VMEM((B,), _i32),
            pltpu.VMEM((B,), _i32),
            pltpu.VMEM((B,), _f32),
            pltpu.VMEM((B, D), _f32),
            pltpu.VMEM_SHARED((NP, D), _f32),
            pltpu.SemaphoreType.DMA,
        ],
    )
    def _agg(tbl_hbm, src_hbm, dst_hbm, nrm_hbm, out_hbm,
             sidx, sidx2, didx, nrm, rows, acc, sem):
        c = lax.axis_index("c")
        s = lax.axis_index("s")

        def _z(i, _):
            for k in range(nvr):
                rows[i, pl.ds(k * L, L)] = jnp.zeros((L,), _f32)
            return 0

        lax.fori_loop(0, B, _z, 0)
        for r in range(RPT):
            pltpu.sync_copy(rows, acc.at[pl.ds((s * RPT + r) * B, B)])
        plsc.subcore_barrier()

        if feat_split:
            base0 = s * CH_FEAT
            off = c * NP
        else:
            base0 = (s * NC + c) * CH_EDGE
            off = 0

        def _body(b, _):
            base = base0 + b * B
            pltpu.sync_copy(src_hbm.at[pl.ds(base, B)], sidx)
            pltpu.sync_copy(dst_hbm.at[pl.ds(base, B)], didx)
            pltpu.sync_copy(nrm_hbm.at[pl.ds(base, B)], nrm)

            def _shift(j, _):
                sl = pl.ds(j * L, L)
                sidx2[sl] = sidx[sl] + off
                return 0

            lax.fori_loop(0, B // L, _shift, 0)
            pltpu.async_copy(tbl_hbm.at[sidx2], rows, sem).wait()

            def _scale(i, _):
                n = plsc.load_gather(nrm, [jnp.broadcast_to(i, (L,))])
                for k in range(nvr):
                    sl = pl.ds(k * L, L)
                    rows[i, sl] = rows[i, sl] * n
                return 0

            lax.fori_loop(0, B, _scale, 0)
            pltpu.sync_copy(rows, acc.at[didx], add=True)
            return 0

        lax.fori_loop(0, nb, _body, 0)
        plsc.subcore_barrier()

        def _wb(r, _):
            rb = (s * RPT + r) * B
            pltpu.sync_copy(acc.at[pl.ds(rb, B)], rows)
            pltpu.sync_copy(rows, out_hbm.at[c, pl.ds(rb, B)])
            return 0

        lax.fori_loop(0, RPT, _wb, 0)

    return _agg


_agg_edge = _make_agg(False)
_agg_feat = _make_agg(True)


_BN = 512
_G = NP // _BN


def _mlp1_body(alo, ahi, xr, d2r, w1r, b1r, w2r, outr):
    a = alo[0] + ahi[0]
    z = a + d2r[:, 0:1] * xr[...]
    h1 = jnp.maximum(jnp.dot(z, w1r[...], preferred_element_type=_f32) + b1r[...], 0.0)
    g = jnp.dot(h1, w2r[...], preferred_element_type=_f32)
    outr[0] = g[:, :128]
    outr[1] = g[:, 128:]


_mlp1 = pl.pallas_call(
    _mlp1_body,
    grid=(_G,),
    in_specs=[
        pl.BlockSpec((1, _BN, 128), lambda i: (0, i, 0)),
        pl.BlockSpec((1, _BN, 128), lambda i: (1, i, 0)),
        pl.BlockSpec((_BN, 128), lambda i: (i, 0)),
        pl.BlockSpec((_BN, 16), lambda i: (i, 0)),
        pl.BlockSpec((128, 512), lambda i: (0, 0)),
        pl.BlockSpec((1, 512), lambda i: (0, 0)),
        pl.BlockSpec((512, 256), lambda i: (0, 0)),
    ],
    out_specs=pl.BlockSpec((NC, _BN, 128), lambda i: (0, i, 0)),
    out_shape=jax.ShapeDtypeStruct((NC, NP, 128), _f32),
)


def _mlp2_body(alo, ahi, glo, ghi, d2r, b2r, wl1r, bl1r, wl2r, bl2r, wl3r, bl3r, outr):
    a = jnp.concatenate([alo[0], ahi[0]], axis=1)
    g = jnp.concatenate([glo[0], ghi[0]], axis=1)
    h = jnp.maximum(a + d2r[:, 0:1] * g + b2r[...], 0.0)
    h = jnp.maximum(jnp.dot(h, wl1r[...], preferred_element_type=_f32) + bl1r[...], 0.0)
    h = jnp.maximum(jnp.dot(h, wl2r[...], preferred_element_type=_f32) + bl2r[...], 0.0)
    lg = jnp.dot(h, wl3r[...], preferred_element_type=_f32) + bl3r[...]
    m = jnp.max(lg, axis=1, keepdims=True)
    e = jnp.exp(lg - m)
    outr[...] = e / jnp.sum(e, axis=1, keepdims=True)


_mlp2 = pl.pallas_call(
    _mlp2_body,
    grid=(_G,),
    in_specs=[
        pl.BlockSpec((1, _BN, 128), lambda i: (0, i, 0)),
        pl.BlockSpec((1, _BN, 128), lambda i: (1, i, 0)),
        pl.BlockSpec((1, _BN, 128), lambda i: (0, i, 0)),
        pl.BlockSpec((1, _BN, 128), lambda i: (1, i, 0)),
        pl.BlockSpec((_BN, 16), lambda i: (i, 0)),
        pl.BlockSpec((1, 256), lambda i: (0, 0)),
        pl.BlockSpec((256, 128), lambda i: (0, 0)),
        pl.BlockSpec((1, 128), lambda i: (0, 0)),
        pl.BlockSpec((128, 64), lambda i: (0, 0)),
        pl.BlockSpec((1, 64), lambda i: (0, 0)),
        pl.BlockSpec((64, 128), lambda i: (0, 0)),
        pl.BlockSpec((1, 128), lambda i: (0, 0)),
    ],
    out_specs=pl.BlockSpec((_BN, 128), lambda i: (i, 0)),
    out_shape=jax.ShapeDtypeStruct((NP, 128), _f32),
)


def kernel(x, edge_index, edge_weight, W1, b1, W2, b2, Wl1, bl1, Wl2, bl2, Wl3, bl3):
    xp = jnp.pad(x, ((0, NP - N), (0, 0)))
    srcp = jnp.pad(edge_index[0], (0, EP - E))
    dstp = jnp.pad(edge_index[1], (0, EP - E))
    ewp = jnp.pad(edge_weight, (0, EP - E))

    dinv, d2 = _deg_dinv(dstp, ewp)
    normp = _norm(srcp, dstp, ewp, dinv)

    agg1 = _agg_edge(xp, srcp, dstp, normp)

    g1s = _mlp1(agg1, agg1, xp, d2, W1, b1.reshape(1, -1), W2)

    agg2 = _agg_feat(g1s.reshape(NC * NP, D), srcp, dstp, normp)

    wl3p = jnp.pad(Wl3, ((0, 0), (0, 128 - Wl3.shape[1])))
    bl3p = jnp.concatenate(
        [bl3, jnp.full((128 - bl3.shape[0],), -1e30, _f32)]
    ).reshape(1, -1)
    probs = _mlp2(
        agg2, agg2, g1s, g1s, d2, b2.reshape(1, -1),
        Wl1, bl1.reshape(1, -1), Wl2, bl2.reshape(1, -1), wl3p, bl3p,
    )
    return probs[:N, : Wl3.shape[1]]
